# baseline (speedup 1.0000x reference)
import functools

import jax
import jax.numpy as jnp
from jax import lax
from jax.experimental import pallas as pl
from jax.experimental.pallas import tpu as pltpu
from jax.experimental.pallas import tpu_sc as plsc

B = 4
S = 4096
D = 2048
NC = 2
NS = 16
NW = NC * NS
S_PER_W = S // NW
C = 16
NCHUNK = S_PER_W // C

_MESH = plsc.VectorSubcoreMesh(core_axis_name="c", subcore_axis_name="s")


@functools.partial(
    pl.kernel,
    out_type=jax.ShapeDtypeStruct((B * S, D), jnp.float32),
    mesh=_MESH,
    scratch_types=[
        pltpu.VMEM((C,), jnp.int32),
        pltpu.VMEM((C, D), jnp.float32),
        pltpu.VMEM((C, D), jnp.float32),
        pltpu.SemaphoreType.DMA,
    ],
)
def _embed(x_hbm, word_hbm, pos_hbm, out_hbm, idx_v, pos_v, acc_v, sem):
    wid = lax.axis_index("s") * NC + lax.axis_index("c")
    s0 = wid * S_PER_W

    @pl.loop(0, NCHUNK)
    def _chunk(ci):
        sbase = s0 + ci * C
        pltpu.sync_copy(pos_hbm.at[pl.ds(sbase, C)], pos_v)

        @pl.loop(0, B)
        def _batch(b):
            flat = b * S + sbase
            pltpu.sync_copy(x_hbm.at[pl.ds(flat, C)], idx_v)
            pltpu.async_copy(word_hbm.at[idx_v], acc_v, sem).wait()

            @pl.loop(0, C)
            def _row(r):
                @pl.loop(0, D, step=16, unroll=8)
                def _col(c0):
                    plsc.addupdate(
                        acc_v.at[r, pl.ds(c0, 16)], pos_v[r, pl.ds(c0, 16)]
                    )

            pltpu.sync_copy(acc_v, out_hbm.at[pl.ds(flat, C)])


def kernel(x, word_table, pos_table):
    out = _embed(x.reshape(B * S), word_table, pos_table)
    return out.reshape(B, S, D)


# speedup vs baseline: 1.1489x; 1.1489x over previous
import functools

import jax
import jax.numpy as jnp
from jax import lax
from jax.experimental import pallas as pl
from jax.experimental.pallas import tpu as pltpu
from jax.experimental.pallas import tpu_sc as plsc

B = 4
S = 4096
D = 2048
NC = 2
NS = 16
NW = NC * NS
S_PER_W = S // NW
C = 16
NCHUNK = S_PER_W // C
NITEMS = NCHUNK * B

_MESH = plsc.VectorSubcoreMesh(core_axis_name="c", subcore_axis_name="s")


@functools.partial(
    pl.kernel,
    out_type=jax.ShapeDtypeStruct((B * S, D), jnp.float32),
    mesh=_MESH,
    scratch_types=[
        pltpu.VMEM((C,), jnp.int32),
        pltpu.VMEM((C,), jnp.int32),
        pltpu.VMEM((C, D), jnp.float32),
        pltpu.VMEM((C, D), jnp.float32),
        pltpu.VMEM((C, D), jnp.float32),
        pltpu.SemaphoreType.DMA,
        pltpu.SemaphoreType.DMA,
    ],
)
def _embed(x_hbm, word_hbm, pos_hbm, out_hbm,
           idx0, idx1, pos_v, acc0, acc1, g0, g1):
    wid = lax.axis_index("s") * NC + lax.axis_index("c")
    s0 = wid * S_PER_W
    idx = (idx0, idx1)
    acc = (acc0, acc1)
    gsem = (g0, g1)

    def flat_base(k):
        ci, b = divmod(k, B)
        return b * S + s0 + ci * C

    def fire_gather(k, slot):
        pltpu.sync_copy(x_hbm.at[pl.ds(flat_base(k), C)], idx[slot])
        pltpu.async_copy(word_hbm.at[idx[slot]], acc[slot], gsem[slot])

    def wait_gather(slot):
        pltpu.make_async_copy(word_hbm.at[idx[slot]], acc[slot],
                              gsem[slot]).wait()

    fire_gather(0, 0)

    for k in range(NITEMS):
        ci, b = divmod(k, B)
        slot = k % 2
        if b == 0:
            pltpu.sync_copy(pos_hbm.at[pl.ds(s0 + ci * C, C)], pos_v)
        if k + 1 < NITEMS:
            fire_gather(k + 1, 1 - slot)
        wait_gather(slot)

        @pl.loop(0, C)
        def _row(r):
            @pl.loop(0, D, step=16, unroll=8)
            def _col(c0):
                plsc.addupdate(
                    acc[slot].at[r, pl.ds(c0, 16)], pos_v[r, pl.ds(c0, 16)]
                )

        pltpu.sync_copy(acc[slot], out_hbm.at[pl.ds(flat_base(k), C)])


def kernel(x, word_table, pos_table):
    out = _embed(x.reshape(B * S), word_table, pos_table)
    return out.reshape(B, S, D)


# speedup vs baseline: 1.1672x; 1.0159x over previous
import functools

import jax
import jax.numpy as jnp
from jax import lax
from jax.experimental import pallas as pl
from jax.experimental.pallas import tpu as pltpu
from jax.experimental.pallas import tpu_sc as plsc

B = 4
S = 4096
D = 2048
NC = 2
NS = 16
NW = NC * NS
S_PER_W = S // NW
C = 16
NCHUNK = S_PER_W // C
NITEMS = NCHUNK * B

_MESH = plsc.VectorSubcoreMesh(core_axis_name="c", subcore_axis_name="s")


@functools.partial(
    pl.kernel,
    out_type=jax.ShapeDtypeStruct((B * S, D), jnp.float32),
    mesh=_MESH,
    scratch_types=[
        pltpu.VMEM((C,), jnp.int32),
        pltpu.VMEM((C,), jnp.int32),
        pltpu.VMEM((C, D), jnp.float32),
        pltpu.VMEM((C, D), jnp.float32),
        pltpu.VMEM((C, D), jnp.float32),
        pltpu.SemaphoreType.DMA,
        pltpu.SemaphoreType.DMA,
        pltpu.SemaphoreType.DMA,
        pltpu.SemaphoreType.DMA,
    ],
)
def _embed(x_hbm, word_hbm, pos_hbm, out_hbm,
           idx0, idx1, pos_v, acc0, acc1, g0, g1, w0, w1):
    wid = lax.axis_index("s") * NC + lax.axis_index("c")
    s0 = wid * S_PER_W
    idx = (idx0, idx1)
    acc = (acc0, acc1)
    gsem = (g0, g1)
    wsem = (w0, w1)

    def flat_base(k):
        ci, b = divmod(k, B)
        return b * S + s0 + ci * C

    def fire_gather(k, slot):
        pltpu.sync_copy(x_hbm.at[pl.ds(flat_base(k), C)], idx[slot])
        pltpu.async_copy(word_hbm.at[idx[slot]], acc[slot], gsem[slot])

    def wait_gather(slot):
        pltpu.make_async_copy(word_hbm.at[idx[slot]], acc[slot],
                              gsem[slot]).wait()

    def fire_write(k, slot):
        pltpu.async_copy(acc[slot], out_hbm.at[pl.ds(flat_base(k), C)],
                         wsem[slot])

    def wait_write(k, slot):
        pltpu.make_async_copy(acc[slot], out_hbm.at[pl.ds(flat_base(k), C)],
                              wsem[slot]).wait()

    fire_gather(0, 0)

    for k in range(NITEMS):
        ci, b = divmod(k, B)
        slot = k % 2
        if b == 0:
            pltpu.sync_copy(pos_hbm.at[pl.ds(s0 + ci * C, C)], pos_v)
        if k + 1 < NITEMS:
            if k >= 1:
                wait_write(k - 1, 1 - slot)
            fire_gather(k + 1, 1 - slot)
        wait_gather(slot)

        @pl.loop(0, C)
        def _row(r):
            @pl.loop(0, D, step=16, unroll=8)
            def _col(c0):
                plsc.addupdate(
                    acc[slot].at[r, pl.ds(c0, 16)], pos_v[r, pl.ds(c0, 16)]
                )

        fire_write(k, slot)

    wait_write(NITEMS - 2, NITEMS % 2)
    wait_write(NITEMS - 1, 1 - NITEMS % 2)


def kernel(x, word_table, pos_table):
    out = _embed(x.reshape(B * S), word_table, pos_table)
    return out.reshape(B, S, D)
